# baseline (speedup 1.0000x reference)
import jax
import jax.numpy as jnp
from jax import lax
from jax.experimental import pallas as pl
from jax.experimental.pallas import tpu as pltpu
from jax.experimental.pallas import tpu_sc as plsc

B, D, H, O, E, FUS, C = 2048, 1024, 512, 512, 8, 1024, 2
BB = 256
NBLK = B // BB
BM = 256
N_PAD = 4096 + E * BM
NB = N_PAD // BM
NBM = 2 * NB
N2 = 2 * N_PAD
NS = 16
TPT = B // NS


def _gate_kernel(xb_ref, xc_ref, gw_ref, gb_ref,
                 e_ref, w_ref, rho_ref, acc_ref):
    i = pl.program_id(0)

    @pl.when(i == 0)
    def _():
        acc_ref[...] = jnp.zeros_like(acc_ref)

    gw = gw_ref[...]
    gb = gb_ref[...]
    iota_e = lax.broadcasted_iota(jnp.int32, (BB, E), 1)
    eyeb = (lax.broadcasted_iota(jnp.int32, (BB, BB), 0) ==
            lax.broadcasted_iota(jnp.int32, (BB, BB), 1)).astype(jnp.float32)
    for m, x_ref in ((0, xb_ref), (1, xc_ref)):
        x = x_ref[...]
        logit = jnp.dot(x, gw, preferred_element_type=jnp.float32) + gb
        mx = jnp.max(logit, axis=1, keepdims=True)
        ex = jnp.exp(logit - mx)
        g = ex / jnp.sum(ex, axis=1, keepdims=True)
        m1 = jnp.max(g, axis=1, keepdims=True)
        i1 = jnp.min(jnp.where(g == m1, iota_e, E), axis=1, keepdims=True)
        mask1 = iota_e == i1
        gm = jnp.where(mask1, -jnp.inf, g)
        m2 = jnp.max(gm, axis=1, keepdims=True)
        i2 = jnp.min(jnp.where(gm == m2, iota_e, E), axis=1, keepdims=True)
        ssum = m1 + m2
        w_ref[m, :, 0:1] = m1 / ssum
        w_ref[m, :, 1:2] = m2 / ssum
        row1 = lax.dot_general(i1.astype(jnp.float32), eyeb,
                               (((0,), (0,)), ((), ())),
                               preferred_element_type=jnp.float32)
        row2 = lax.dot_general(i2.astype(jnp.float32), eyeb,
                               (((0,), (0,)), ((), ())),
                               preferred_element_type=jnp.float32)
        e_ref[m, 0:1, :] = row1.astype(jnp.int32)
        e_ref[m, 1:2, :] = row2.astype(jnp.int32)
        acc_ref[...] += (1.0 if m == 0 else 2.0) * jnp.sum(
            g, axis=0, keepdims=True)
    rho_ref[...] = acc_ref[...]


def _gate(xb, xc, gate_w, gate_b):
    return pl.pallas_call(
        _gate_kernel,
        grid=(NBLK,),
        in_specs=[
            pl.BlockSpec((BB, D), lambda i: (i, 0)),
            pl.BlockSpec((BB, D), lambda i: (i, 0)),
            pl.BlockSpec((D, E), lambda i: (0, 0)),
            pl.BlockSpec((1, E), lambda i: (0, 0)),
        ],
        out_specs=[
            pl.BlockSpec((2, 2, BB), lambda i: (0, 0, i)),
            pl.BlockSpec((2, BB, 2), lambda i: (0, i, 0)),
            pl.BlockSpec((1, E), lambda i: (0, 0)),
        ],
        out_shape=[
            jax.ShapeDtypeStruct((2, 2, B), jnp.int32),
            jax.ShapeDtypeStruct((2, B, 2), jnp.float32),
            jax.ShapeDtypeStruct((1, E), jnp.float32),
        ],
        scratch_shapes=[pltpu.VMEM((1, E), jnp.float32)],
        compiler_params=pltpu.CompilerParams(
            dimension_semantics=("arbitrary",)),
    )(xb, xc, gate_w, gate_b)


def _splat(v, i):
    return v.at[jnp.zeros((16,), jnp.int32) + i].get(mode="promise_in_bounds")


def _gat(v, idx):
    return v.at[idx].get(mode="promise_in_bounds")


def _dispatch_body(e_hbm, x_hbm,
                   xs_hbm, pos_hbm, cnt_hbm, be_hbm, nv_hbm, dbg_hbm,
                   ebuf, histbuf, xrows, slotbuf,
                   vb, bebuf, nvbuf):
    c = lax.axis_index("c")
    s = lax.axis_index("s")
    m = c
    tb = s * TPT
    moffs = m * N_PAD
    lane = lax.broadcasted_iota(jnp.int32, (16,), 0)
    z = jnp.zeros((16,), jnp.int32)
    ones = z + 1

    def vreg_hist(ev):
        hv = z
        for l in range(16):
            evl = _splat(ev, l)
            hv = hv + jnp.where(lane == evl, ones, z)
        return hv

    pltpu.sync_copy(e_hbm.at[m, :, pl.ds(tb, TPT)], ebuf)
    evs = []
    hvs = []
    hist = z
    for k in range(2):
        for j in range(TPT // 16):
            ev = ebuf[k, pl.ds(j * 16, 16)]
            hv = vreg_hist(ev)
            evs.append(ev)
            hvs.append(hv)
            hist = hist + hv
    vb[...] = hist
    pltpu.sync_copy(vb, dbg_hbm.at[m, s])
    plsc.subcore_barrier()

    pltpu.sync_copy(dbg_hbm.at[m], histbuf)
    pre = z
    tot = z
    svec = z + s
    for t in range(NS):
        ht = histbuf[t, :]
        pre = pre + ht * jnp.clip(svec - t, 0, 1)
        tot = tot + ht
    padded = jnp.bitwise_and(tot + (BM - 1), -BM)
    inc = padded
    for sh in (1, 2, 4, 8):
        g = _gat(inc, jnp.maximum(lane - sh, z))
        inc = inc + jnp.where(lane >= sh, g, z)
    base = inc - padded
    totpad = _splat(inc, 15)

    @pl.when(s == 0)
    def _tile0():
        vb[...] = tot
        pltpu.sync_copy(vb, cnt_hbm.at[m])
        end = base + padded
        for v in range(2):
            jv = lane + 16 * v
            bs = jv * BM
            eof = z
            for e in range(E):
                end_e = _splat(end, e)
                eof = eof + jnp.where(bs >= end_e, ones, z)
            eofc = jnp.minimum(eof, E - 1)
            base_of = _gat(base, eofc)
            cnt_of = _gat(tot, eofc)
            usedi = jnp.where(bs < totpad, ones, z)
            nv = jnp.clip(cnt_of - (bs - base_of), 0, BM)
            nv = nv * usedi
            bev = eofc * usedi
            bebuf[pl.ds(v * 16, 16)] = bev
            nvbuf[pl.ds(v * 16, 16)] = nv
        pltpu.sync_copy(bebuf, be_hbm.at[m])
        pltpu.sync_copy(nvbuf, nv_hbm.at[m])

    run = base + pre
    for k in range(2):
        for j in range(TPT // 16):
            vi = k * (TPT // 16) + j
            ev = evs[vi]
            bases = _gat(run, ev) + moffs
            rank = z
            for sh in range(1, 16):
                g = _gat(ev, jnp.maximum(lane - sh, z))
                rank = rank + jnp.where(lane >= sh,
                                        jnp.where(g == ev, ones, z), z)
            slot = bases + rank
            slotbuf[k, j // 4, pl.ds((j % 4) * 16, 16)] = slot
            run = run + hvs[vi]

    for k in range(2):
        for h in range(2):
            pltpu.sync_copy(slotbuf.at[k, h],
                            pos_hbm.at[2 * m + k, pl.ds(tb + h * 64, 64)])
    for h in range(2):
        pltpu.sync_copy(x_hbm.at[m, pl.ds(tb + h * 64, 64), :], xrows)
        for k in range(2):
            pltpu.sync_copy(xrows, xs_hbm.at[slotbuf.at[k, h]])


def _dispatch(e_all, x_all):
    f = pl.kernel(
        _dispatch_body,
        out_type=[
            jax.ShapeDtypeStruct((N2, D), jnp.float32),
            jax.ShapeDtypeStruct((4, B), jnp.int32),
            jax.ShapeDtypeStruct((2, 16), jnp.int32),
            jax.ShapeDtypeStruct((2, 32), jnp.int32),
            jax.ShapeDtypeStruct((2, 32), jnp.int32),
            jax.ShapeDtypeStruct((2, NS, 16), jnp.int32),
        ],
        mesh=plsc.VectorSubcoreMesh(core_axis_name="c", subcore_axis_name="s"),
        scratch_types=[
            pltpu.VMEM((2, TPT), jnp.int32),
            pltpu.VMEM((NS, 16), jnp.int32),
            pltpu.VMEM((64, D), jnp.float32),
            pltpu.VMEM((2, 2, 64), jnp.int32),
            pltpu.VMEM((16,), jnp.int32),
            pltpu.VMEM((32,), jnp.int32),
            pltpu.VMEM((32,), jnp.int32),
        ],
    )
    return f(e_all, x_all)


def _mm_kernel(be_r, nv_r, xs_ref, W1_ref, b1_ref, W2_ref, b2_ref,
               eo_ref, esum_ref, acc_ref):
    i = pl.program_id(0)

    @pl.when(i == 0)
    def _():
        acc_ref[...] = jnp.zeros_like(acc_ref)

    nv = nv_r[i]
    fac = jnp.where(i >= NB, 2.0, 1.0)

    @pl.when(nv > 0)
    def _compute():
        x = xs_ref[...]
        h = jnp.maximum(
            jnp.dot(x, W1_ref[0], preferred_element_type=jnp.float32)
            + b1_ref[0], 0.0)
        eo = (jnp.dot(h, W2_ref[0], preferred_element_type=jnp.float32)
              + b2_ref[0])
        rmask = lax.broadcasted_iota(jnp.int32, (BM, 1), 0) < nv
        eo_m = jnp.where(rmask, eo, 0.0)
        eo_ref[...] = eo_m
        vs = jnp.sum(eo_m, axis=0, keepdims=True)
        ohc = (lax.broadcasted_iota(jnp.int32, (E, 1), 0)
               == be_r[i]).astype(jnp.float32)
        acc_ref[...] += fac * lax.dot_general(
            ohc, vs, (((1,), (0,)), ((), ())),
            preferred_element_type=jnp.float32)

    @pl.when(nv == 0)
    def _skip():
        eo_ref[...] = jnp.zeros((BM, O), jnp.float32)

    esum_ref[...] = acc_ref[...]


def _mm(be, nv, xs, W1, b1, W2, b2):
    grid_spec = pltpu.PrefetchScalarGridSpec(
        num_scalar_prefetch=2,
        grid=(NBM,),
        in_specs=[
            pl.BlockSpec((BM, D), lambda i, be_r, nv_r: (i, 0)),
            pl.BlockSpec((1, D, H), lambda i, be_r, nv_r: (be_r[i], 0, 0)),
            pl.BlockSpec((1, 1, H), lambda i, be_r, nv_r: (be_r[i], 0, 0)),
            pl.BlockSpec((1, H, O), lambda i, be_r, nv_r: (be_r[i], 0, 0)),
            pl.BlockSpec((1, 1, O), lambda i, be_r, nv_r: (be_r[i], 0, 0)),
        ],
        out_specs=[
            pl.BlockSpec((BM, O), lambda i, be_r, nv_r: (i, 0)),
            pl.BlockSpec((E, O), lambda i, be_r, nv_r: (0, 0)),
        ],
        scratch_shapes=[pltpu.VMEM((E, O), jnp.float32)],
    )
    return pl.pallas_call(
        _mm_kernel,
        grid_spec=grid_spec,
        out_shape=[
            jax.ShapeDtypeStruct((N2, O), jnp.float32),
            jax.ShapeDtypeStruct((E, O), jnp.float32),
        ],
        compiler_params=pltpu.CompilerParams(
            dimension_semantics=("arbitrary",)),
    )(be, nv, xs, W1, b1.reshape(E, 1, H), W2, b2.reshape(E, 1, O))


def _gather_body(pos_hbm, eo_hbm, r_hbm, idx2, rows, sem):
    c = lax.axis_index("c")
    s = lax.axis_index("s")
    wid = s * 2 + c
    q = wid // 8
    b0 = (wid % 8) * 256
    for h in range(2):
        pltpu.sync_copy(pos_hbm.at[q, pl.ds(b0 + h * 128, 128)], idx2.at[h])
    for h in range(2):
        pltpu.async_copy(eo_hbm.at[idx2.at[h]], rows, sem).wait()
        pltpu.sync_copy(rows, r_hbm.at[pl.ds(wid * 256 + h * 128, 128), :])


def _gatherr(pos_all, eo):
    f = pl.kernel(
        _gather_body,
        out_type=[jax.ShapeDtypeStruct((4 * B, O), jnp.float32)],
        mesh=plsc.VectorSubcoreMesh(core_axis_name="c", subcore_axis_name="s"),
        scratch_types=[
            pltpu.VMEM((2, 128), jnp.int32),
            pltpu.VMEM((128, O), jnp.float32),
            pltpu.SemaphoreType.DMA,
        ],
    )
    return f(pos_all, eo)[0]


def _fusion_kernel(r0_ref, r1_ref, r2_ref, r3_ref, w_ref, esum_ref, cnt_ref,
                   rho_ref, fw_ref, fb_ref, cw_ref, cb_ref,
                   out_ref, dist_ref, eq_ref):
    wb = w_ref[0]
    wc = w_ref[1]
    fin_b = wb[:, 0:1] * r0_ref[...] + wb[:, 1:2] * r1_ref[...]
    fin_c = wc[:, 0:1] * r2_ref[...] + wc[:, 1:2] * r3_ref[...]
    f0 = fw_ref[0:O, :]
    f12 = fw_ref[O:2 * O, :] + fw_ref[2 * O:3 * O, :]
    fused = jnp.maximum(
        jnp.dot(fin_b, f0, preferred_element_type=jnp.float32)
        + jnp.dot(fin_c, f12, preferred_element_type=jnp.float32)
        + fb_ref[...], 0.0)
    out_ref[...] = (jnp.dot(fused, cw_ref[...],
                            preferred_element_type=jnp.float32) + cb_ref[...])

    cnt_row = (cnt_ref[0:1, 0:E] + 2 * cnt_ref[1:2, 0:E]).astype(jnp.float32)
    eq_ref[0, 0] = jnp.sum(cnt_row * rho_ref[...]) * (1.0 / E)
    eyee = (lax.broadcasted_iota(jnp.int32, (E, E), 0) ==
            lax.broadcasted_iota(jnp.int32, (E, E), 1))
    cnt_col = lax.dot_general(eyee.astype(jnp.float32), cnt_row,
                              (((1,), (1,)), ((), ())),
                              preferred_element_type=jnp.float32)
    avg = esum_ref[...] / jnp.maximum(cnt_col, 1.0)
    G = lax.dot_general(avg, avg, (((1,), (1,)), ((), ())),
                        preferred_element_type=jnp.float32)
    Gd = jnp.where(eyee, G, 0.0)
    diag_c = jnp.sum(Gd, axis=1, keepdims=True)
    diag_r = jnp.sum(Gd, axis=0, keepdims=True)
    d2 = diag_c + diag_r - 2.0 * G
    sim = jnp.exp(-0.5 * d2)
    pm = (~eyee) & (cnt_col > 0.0) & (cnt_row > 0.0)
    npairs = jnp.sum(jnp.where(pm, 1.0, 0.0)) * 0.5
    ssum = jnp.sum(jnp.where(pm, sim, 0.0)) * 0.5
    dist_ref[0, 0] = -ssum / jnp.maximum(npairs, 1.0)


def _fusion(r, w_all, esum, cnt_all, rho_hat, fus_w, fus_b, cls_w, cls_b):
    def rspec(q):
        return pl.BlockSpec((BB, O), lambda i, q=q: (q * NBLK + i, 0))
    return pl.pallas_call(
        _fusion_kernel,
        grid=(NBLK,),
        in_specs=[
            rspec(0), rspec(1), rspec(2), rspec(3),
            pl.BlockSpec((2, BB, 2), lambda i: (0, i, 0)),
            pl.BlockSpec((E, O), lambda i: (0, 0)),
            pl.BlockSpec((2, 16), lambda i: (0, 0)),
            pl.BlockSpec((1, E), lambda i: (0, 0)),
            pl.BlockSpec((3 * O, FUS), lambda i: (0, 0)),
            pl.BlockSpec((1, FUS), lambda i: (0, 0)),
            pl.BlockSpec((FUS, C), lambda i: (0, 0)),
            pl.BlockSpec((1, C), lambda i: (0, 0)),
        ],
        out_specs=[
            pl.BlockSpec((BB, C), lambda i: (i, 0)),
            pl.BlockSpec((1, 1), lambda i: (0, 0), memory_space=pltpu.SMEM),
            pl.BlockSpec((1, 1), lambda i: (0, 0), memory_space=pltpu.SMEM),
        ],
        out_shape=[
            jax.ShapeDtypeStruct((B, C), jnp.float32),
            jax.ShapeDtypeStruct((1, 1), jnp.float32),
            jax.ShapeDtypeStruct((1, 1), jnp.float32),
        ],
        compiler_params=pltpu.CompilerParams(
            dimension_semantics=("arbitrary",)),
    )(r, r, r, r, w_all, esum, cnt_all, rho_hat, fus_w, fus_b, cls_w, cls_b)


def kernel(vec_binary, vec_cfg, vec_fcg, W1, b1, W2, b2, gate_w, gate_b,
           fus_w, fus_b, cls_w, cls_b):
    del vec_fcg
    e_all, w_all, rho_hat = _gate(vec_binary, vec_cfg, gate_w,
                                  gate_b.reshape(1, E))
    x_all = jnp.stack([vec_binary, vec_cfg], axis=0)
    xs, pos_all, cnt_all, be2, nv2, _hist = _dispatch(e_all, x_all)
    be = jnp.concatenate([be2[0, :NB], be2[1, :NB]])
    nv = jnp.concatenate([nv2[0, :NB], nv2[1, :NB]])
    eo, esum = _mm(be, nv, xs, W1, b1, W2, b2)
    r = _gatherr(pos_all, eo)
    out, dist, eq = _fusion(r, w_all, esum, cnt_all, rho_hat, fus_w,
                            fus_b.reshape(1, FUS), cls_w, cls_b.reshape(1, C))
    return out, dist.reshape(()), eq.reshape(())
